# baseline (speedup 1.0000x reference)
import jax
import jax.numpy as jnp
from jax.experimental import pallas as pl
from jax.experimental.pallas import tpu as pltpu

_CHUNK = 4000
_NBUF = 8
_NCORES = 1


def _mlp_body(x_hbm, w1_ref, b1_ref, w2_ref, b2_ref, o_hbm,
              x_buf, o_buf, in_sems, out_sems):
    n = x_hbm.shape[0]
    rows_per_core = n // _NCORES
    nchunks = rows_per_core // _CHUNK
    base = 0

    half = _CHUNK // 2

    def in_copies(i, slot):
        return [
            pltpu.make_async_copy(
                x_hbm.at[pl.ds(base + i * _CHUNK + q * half, half), :],
                x_buf.at[slot, pl.ds(q * half, half), :],
                in_sems.at[slot, q],
            )
            for q in range(2)
        ]

    def out_copies(i, slot):
        return [
            pltpu.make_async_copy(
                o_buf.at[slot, pl.ds(q * half, half), :],
                o_hbm.at[pl.ds(base + i * _CHUNK + q * half, half), :],
                out_sems.at[slot, q],
            )
            for q in range(2)
        ]

    def start_all(copies):
        for c in copies:
            c.start()

    def wait_all(copies):
        for c in copies:
            c.wait()

    for k in range(_NBUF):
        start_all(in_copies(k, k))

    def step(i, carry):
        slot = jax.lax.rem(i, _NBUF)

        @pl.when(i >= _NBUF)
        def _():
            wait_all(out_copies(i - _NBUF, slot))

        ins = in_copies(i, slot)
        outs = out_copies(i, slot)
        for q in range(2):
            ins[q].wait()
            xq = x_buf[slot, pl.ds(q * half, half), :]
            h = jnp.tanh(
                jnp.dot(
                    xq.astype(jnp.bfloat16),
                    w1_ref[...],
                    preferred_element_type=jnp.float32,
                )
                + b1_ref[...]
            )
            o_buf[slot, pl.ds(q * half, half), :] = jnp.tanh(
                jnp.dot(
                    h.astype(jnp.bfloat16),
                    w2_ref[...],
                    preferred_element_type=jnp.float32,
                )
                + b2_ref[...]
            )
            outs[q].start()

        @pl.when(i + _NBUF < nchunks)
        def _():
            start_all(in_copies(i + _NBUF, slot))

        return carry

    jax.lax.fori_loop(0, nchunks, step, 0)

    for k in range(nchunks - _NBUF, nchunks):
        wait_all(out_copies(k, k % _NBUF))


def kernel(x, W1, b1, W2, b2):
    n, hid = x.shape
    return pl.pallas_call(
        _mlp_body,
        in_specs=[
            pl.BlockSpec(memory_space=pl.ANY),
            pl.BlockSpec(memory_space=pltpu.MemorySpace.VMEM),
            pl.BlockSpec(memory_space=pltpu.MemorySpace.VMEM),
            pl.BlockSpec(memory_space=pltpu.MemorySpace.VMEM),
            pl.BlockSpec(memory_space=pltpu.MemorySpace.VMEM),
        ],
        out_specs=pl.BlockSpec(memory_space=pl.ANY),
        out_shape=jax.ShapeDtypeStruct((n, hid), jnp.float32),
        scratch_shapes=[
            pltpu.VMEM((_NBUF, _CHUNK, hid), jnp.float32),
            pltpu.VMEM((_NBUF, _CHUNK, hid), jnp.float32),
            pltpu.SemaphoreType.DMA((_NBUF, 2)),
            pltpu.SemaphoreType.DMA((_NBUF, 2)),
        ],
    )(
        x,
        W1.T.astype(jnp.bfloat16),
        b1.reshape(1, hid),
        W2.T.astype(jnp.bfloat16),
        b2.reshape(1, hid),
    )


# speedup vs baseline: 1.3626x; 1.3626x over previous
import jax
import jax.numpy as jnp
from jax.experimental import pallas as pl
from jax.experimental.pallas import tpu as pltpu

_CHUNK = 4000
_NBUF = 8


def _mlp_body(x_hbm, w1_ref, b1_ref, w2_ref, b2_ref, o_hbm,
              x_buf, o_buf, in_sems, out_sems):
    n = x_hbm.shape[0]
    nchunks = n // _CHUNK

    def in_copy(i, slot):
        return pltpu.make_async_copy(
            x_hbm.at[pl.ds(i * _CHUNK, _CHUNK), :],
            x_buf.at[slot],
            in_sems.at[slot],
        )

    def out_copy(i, slot):
        return pltpu.make_async_copy(
            o_buf.at[slot],
            o_hbm.at[pl.ds(i * _CHUNK, _CHUNK), :],
            out_sems.at[slot],
        )

    for k in range(_NBUF):
        in_copy(k, k).start()

    def step(i, carry):
        slot = jax.lax.rem(i, _NBUF)

        @pl.when(i >= _NBUF)
        def _():
            out_copy(i - _NBUF, slot).wait()

        in_copy(i, slot).wait()

        h = jnp.tanh(
            jnp.dot(
                x_buf[slot].astype(jnp.bfloat16),
                w1_ref[...],
                preferred_element_type=jnp.float32,
            )
            + b1_ref[...]
        )
        o_buf[slot] = jnp.tanh(
            jnp.dot(
                h.astype(jnp.bfloat16),
                w2_ref[...],
                preferred_element_type=jnp.float32,
            )
            + b2_ref[...]
        )

        out_copy(i, slot).start()

        @pl.when(i + _NBUF < nchunks)
        def _():
            in_copy(i + _NBUF, slot).start()

        return carry

    jax.lax.fori_loop(0, nchunks, step, 0)

    for k in range(nchunks - _NBUF, nchunks):
        out_copy(k, k % _NBUF).wait()


def kernel(x, W1, b1, W2, b2):
    n, hid = x.shape
    return pl.pallas_call(
        _mlp_body,
        in_specs=[
            pl.BlockSpec(memory_space=pl.ANY),
            pl.BlockSpec(memory_space=pltpu.MemorySpace.VMEM),
            pl.BlockSpec(memory_space=pltpu.MemorySpace.VMEM),
            pl.BlockSpec(memory_space=pltpu.MemorySpace.VMEM),
            pl.BlockSpec(memory_space=pltpu.MemorySpace.VMEM),
        ],
        out_specs=pl.BlockSpec(memory_space=pl.ANY),
        out_shape=jax.ShapeDtypeStruct((n, hid), jnp.float32),
        scratch_shapes=[
            pltpu.VMEM((_NBUF, _CHUNK, hid), jnp.float32),
            pltpu.VMEM((_NBUF, _CHUNK, hid), jnp.float32),
            pltpu.SemaphoreType.DMA((_NBUF,)),
            pltpu.SemaphoreType.DMA((_NBUF,)),
        ],
        compiler_params=pltpu.CompilerParams(
            disable_semaphore_checks=True,
        ),
    )(
        x,
        W1.T.astype(jnp.bfloat16),
        b1.reshape(1, hid),
        W2.T.astype(jnp.bfloat16),
        b2.reshape(1, hid),
    )
